# baseline (speedup 1.0000x reference)
import jax
import jax.numpy as jnp
from jax import lax
from jax.experimental import pallas as pl
from jax.experimental.pallas import tpu as pltpu
from jax.experimental.pallas import tpu_sc as plsc

N = 10000
E = 160000
D = 256
H = 128
NC = 2
NS = 16
L = 16
K = 96
CH = 108
CHA = CH + 3
EPT = CHA * K
RREC = 2 + K // 6
RPT = N // NS
WOFF = (0, 96, 192, 288, 384, 480, 529)
WCH = len(WOFF)
Q = 2.0 ** 17
QINV = 2.0 ** -17


def _mm_body(x_ref, w_ref, o_ref):
    o_ref[...] = jnp.dot(x_ref[...], w_ref[...],
                         preferred_element_type=jnp.float32)


def _matmul(x, weight):
    bn = 1000
    return pl.pallas_call(
        _mm_body,
        grid=(N // bn,),
        in_specs=[
            pl.BlockSpec((bn, D), lambda i: (i, 0)),
            pl.BlockSpec((D, D), lambda i: (0, 0)),
        ],
        out_specs=pl.BlockSpec((bn, D), lambda i: (i, 0)),
        out_shape=jax.ShapeDtypeStruct((N, D), jnp.float32),
    )(x, weight)


def _sc_body(xw2_hbm, rec_hbm, widx_hbm, out_hbm,
             r0, r1, r2, r3, bufa, bufb, widx_v, acc,
             rs0, rs1, rs2, rs3, gsa, gsb, sa, sb, wsa, wsb):
    c = lax.axis_index("c")
    s = lax.axis_index("s")
    base = s * RPT
    rr_ = (r0, r1, r2, r3)
    rs_ = (rs0, rs1, rs2, rs3)
    bb_ = (bufa, bufb)
    gs_ = (gsa, gsb)
    ss_ = (sa, sb)

    def rec_start(j, slot):
        pltpu.async_copy(rec_hbm.at[c, s, j], rr_[slot], rs_[slot])

    def rec_wait(slot):
        pltpu.make_async_copy(rec_hbm.at[c, s, 0], rr_[slot],
                              rs_[slot]).wait()

    def g_start(slot, p):
        pltpu.async_copy(xw2_hbm.at[rr_[slot].at[0]], bb_[p], gs_[p])

    def g_wait(p):
        pltpu.make_async_copy(xw2_hbm.at[pl.ds(0, K)], bb_[p],
                              gs_[p]).wait()

    def s_start(slot, p):
        pltpu.async_copy(bb_[p], acc.at[rr_[slot].at[1]], ss_[p], add=True)

    def s_wait(p):
        pltpu.make_async_copy(bb_[p], acc.at[rr_[0].at[1]], ss_[p]).wait()

    rec_start(0, 0)
    rec_start(1, 1)
    rec_start(2, 2)
    pltpu.async_copy(widx_hbm.at[c, s], widx_v, wsb)
    rec_wait(0)
    g_start(0, 0)

    zeros = jnp.zeros((L,), jnp.float32)

    def zrow(r, _):
        for v in range(H // L):
            bufb[r, pl.ds(v * L, L)] = zeros
        return 0

    lax.fori_loop(0, K, zrow, 0)
    for k in range(WCH):
        pltpu.async_copy(bufb, acc.at[pl.ds(base + WOFF[k], K)], wsa)
    for k in range(WCH):
        pltpu.make_async_copy(bufb, acc.at[pl.ds(base, K)], wsa).wait()
    pltpu.make_async_copy(widx_hbm.at[c, s], widx_v, wsb).wait()
    plsc.subcore_barrier()

    pltpu.async_copy(bufb, acc.at[rr_[0].at[1]], sa, add=True)
    pltpu.async_copy(bufb, acc.at[rr_[0].at[1]], sb, add=True)

    def scale(p, slot):
        buf = bb_[p]
        rec = rr_[slot]

        def rowloop(rr, _):
            for u in range(6):
                e = rr * 6 + u
                w = rec[2 + rr, pl.ds(u * L, L)].astype(jnp.float32) * QINV
                for v in range(H // L):
                    sl = pl.ds(v * L, L)
                    buf[e, sl] = buf[e, sl] * w
            return 0

        lax.fori_loop(0, K // 6, rowloop, 0)

    T = CH // 4

    def pipe(t, _):
        j = 4 * t
        for o in range(4):
            so, s1, s3 = o, (o + 1) % 4, (o + 3) % 4
            po, p1 = o % 2, (o + 1) % 2
            rec_wait(s1)
            s_wait(p1)
            rec_start(j + o + 3, s3)
            g_start(s1, p1)
            g_wait(po)
            scale(po, so)
            s_start(so, po)
        return 0

    lax.fori_loop(0, T, pipe, 0)
    g_wait(0)
    s_wait(1)
    rec_wait(1)
    rec_wait(2)
    plsc.subcore_barrier()

    def wo(k, p, wsem):
        buf = bb_[p]
        pltpu.sync_copy(acc.at[pl.ds(base + WOFF[k], K)], buf)

        def rrow(r, _):
            for v in range(H // L):
                sl = pl.ds(v * L, L)
                buf[r, sl] = jnp.maximum(buf[r, sl], 0.0)
            return 0

        lax.fori_loop(0, K, rrow, 0)
        pltpu.async_copy(buf, out_hbm.at[widx_v.at[k]], wsem)

    def w_wait(p, wsem):
        pltpu.make_async_copy(bb_[p], out_hbm.at[widx_v.at[0]],
                              wsem).wait()

    ws_ = (wsa, wsb)
    wo(0, 0, wsa)
    wo(1, 1, wsb)
    for k in range(2, WCH):
        w_wait(k % 2, ws_[k % 2])
        wo(k, k % 2, ws_[k % 2])
    w_wait(0, wsa)
    w_wait(1, wsb)


def _sc_scatter(xw2, rec, widx):
    mesh = plsc.VectorSubcoreMesh(core_axis_name="c", subcore_axis_name="s",
                                  num_cores=NC, num_subcores=NS)
    dma = pltpu.SemaphoreType.DMA
    return pl.kernel(
        _sc_body,
        out_type=jax.ShapeDtypeStruct((2 * N, H), jnp.float32),
        mesh=mesh,
        scratch_types=[
            pltpu.VMEM((RREC, K), jnp.int32),
            pltpu.VMEM((RREC, K), jnp.int32),
            pltpu.VMEM((RREC, K), jnp.int32),
            pltpu.VMEM((RREC, K), jnp.int32),
            pltpu.VMEM((K, H), jnp.float32),
            pltpu.VMEM((K, H), jnp.float32),
            pltpu.VMEM((WCH, K), jnp.int32),
            pltpu.VMEM_SHARED((N, H), jnp.float32),
            dma, dma, dma, dma, dma, dma, dma, dma, dma, dma,
        ],
    )(xw2, rec, widx)


def kernel(x, edge_index, edge_weight, weight):
    xw = _matmul(x, weight)
    xw2 = xw.reshape(2 * N, H)

    ept = E // NS
    tpad = ((0, 0), (0, EPT - ept))
    srcp = jnp.pad(edge_index[0].reshape(NS, ept), tpad)
    dstp = jnp.pad(edge_index[1].reshape(NS, ept), tpad)
    ewp = jnp.pad(edge_weight.reshape(NS, ept), tpad)

    core = jnp.arange(NC, dtype=jnp.int32)
    src_c = (2 * srcp[None] + core[:, None, None]).reshape(NC, NS, CHA, 1, K)
    dst_c = jnp.broadcast_to(dstp.reshape(1, NS, CHA, 1, K),
                             (NC, NS, CHA, 1, K))
    ew_q = jnp.round(ewp * Q).astype(jnp.int32).reshape(NS, CHA, K // 6, 6)
    ew_rep = jnp.broadcast_to(ew_q[..., None],
                              (NS, CHA, K // 6, 6, L)).reshape(
                                  NS, CHA, K // 6, K)
    ew_bits = jnp.broadcast_to(ew_rep[None], (NC, NS, CHA, K // 6, K))
    rec = jnp.concatenate([src_c, dst_c, ew_bits], axis=3)

    offs = jnp.array(WOFF, jnp.int32)
    rows = (jnp.arange(NS, dtype=jnp.int32)[:, None, None] * RPT
            + offs[None, :, None]
            + jnp.arange(K, dtype=jnp.int32)[None, None, :])
    widx = 2 * rows[None] + core[:, None, None, None]

    out_flat = _sc_scatter(xw2, rec, widx)
    return out_flat.reshape(N, D)
